# baseline (speedup 1.0000x reference)
import jax
import jax.numpy as jnp
import numpy as np
from jax.experimental import pallas as pl
from jax.experimental.pallas import tpu as pltpu

_PERM_NP = np.array([0, 2, 6, 3, 7, 1, 5, 8, 4])
_PERM = jnp.array(_PERM_NP)
_INV_PERM = jnp.array(np.argsort(_PERM_NP))
_GATE_EXPAND = jnp.array([0, 0, 0, 1, 1, 1, 1, 1])
_L_EXPAND_NP = np.array([0, 1, 1, 1, 2, 2, 2, 2, 2])
_LMAX = 2
_NCOEFF = 9


def _smooth_leaky(x, alpha=0.2):
    return ((1 + alpha) / 2.0) * x + ((1 - alpha) / 2.0) * x * (2.0 * jax.nn.sigmoid(x) - 1.0)


def _so2_conv(emb, w0, b0, w1, w2, extra):
    E, _, C = emb.shape
    em = emb[:, _PERM]
    x0 = em[:, 0:3].reshape(E, 3 * C) @ w0 + b0
    if extra > 0:
        x0_extra = x0[:, :extra]
        x0 = x0[:, extra:]
    else:
        x0_extra = None
    cout = w1.shape[1] // 4
    out0 = x0.reshape(E, 3, cout)
    x1 = em[:, 3:7].reshape(E, 2, 2 * C)
    y1 = x1 @ w1
    h1 = y1.shape[-1] // 2
    y1r, y1i = y1[..., :h1], y1[..., h1:]
    m1r = y1r[:, 0] - y1i[:, 1]
    m1i = y1r[:, 1] + y1i[:, 0]
    out1 = jnp.stack([m1r, m1i], axis=1).reshape(E, 4, cout)
    x2 = em[:, 7:9].reshape(E, 2, C)
    y2 = x2 @ w2
    h2 = y2.shape[-1] // 2
    y2r, y2i = y2[..., :h2], y2[..., h2:]
    m2r = y2r[:, 0] - y2i[:, 1]
    m2i = y2r[:, 1] + y2i[:, 0]
    out2 = jnp.stack([m2r, m2i], axis=1).reshape(E, 2, cout)
    out = jnp.concatenate([out0, out1, out2], axis=1)
    return out[:, _INV_PERM], x0_extra


def _proj_body(node_ref, pw_ref, pb_ref, out_ref):
    m = pl.program_id(0)
    acc = jax.lax.dot_general(
        node_ref[0], pw_ref[0],
        dimension_numbers=(((1,), (1,)), ((), ())),
        preferred_element_type=jnp.float32)

    @pl.when(m == 0)
    def _():
        out_ref[0] = acc + pb_ref[:][None, :]

    @pl.when(m != 0)
    def _():
        out_ref[0] = acc


def _so3_project(node, proj_w, proj_b):
    N, M, C = node.shape
    OUT = proj_w.shape[1]
    node_t = jnp.transpose(node, (1, 0, 2))
    pw = proj_w[_L_EXPAND_NP]
    out_t = pl.pallas_call(
        _proj_body,
        grid=(M,),
        in_specs=[
            pl.BlockSpec((1, N, C), lambda m: (m, 0, 0)),
            pl.BlockSpec((1, OUT, C), lambda m: (m, 0, 0)),
            pl.BlockSpec((OUT,), lambda m: (0,)),
        ],
        out_specs=pl.BlockSpec((1, N, OUT), lambda m: (m, 0, 0)),
        out_shape=jax.ShapeDtypeStruct((M, N, OUT), jnp.float32),
    )(node_t, pw, proj_b)
    return jnp.transpose(out_t, (1, 0, 2))


def kernel(x, atomic_numbers, edge_distance, edge_index, edge_fea, wigner, wigner_inv,
           src_emb, tgt_emb, w0_1, b0_1, w1_1, w2_1, ln_gamma, ln_beta, alpha_dot,
           w0_2, b0_2, w1_2, w2_2, proj_w, proj_b):
    N = x.shape[0]
    E = edge_index.shape[1]
    NH, AC, VC, HID = 4, 16, 8, 32
    src, dst = edge_index[0], edge_index[1]
    msg = jnp.concatenate([x[src], x[dst], edge_fea], axis=2)
    msg = jnp.einsum('eij,ejc->eic', wigner, msg)
    msg, x0_extra = _so2_conv(msg, w0_1, b0_1, w1_1, w2_1, NH * AC + _LMAX * HID)
    x0_alpha = x0_extra[:, :NH * AC]
    x0_gate = x0_extra[:, NH * AC:]
    gates = jax.nn.sigmoid(x0_gate).reshape(E, _LMAX, HID)[:, _GATE_EXPAND]
    msg = jnp.concatenate([jax.nn.silu(msg[:, 0:1]), msg[:, 1:] * gates], axis=1)
    msg, _ = _so2_conv(msg, w0_2, b0_2, w1_2, w2_2, 0)
    a = x0_alpha.reshape(E, NH, AC)
    mu = a.mean(axis=-1, keepdims=True)
    var = ((a - mu) ** 2).mean(axis=-1, keepdims=True)
    a = (a - mu) / jnp.sqrt(var + 1e-5) * ln_gamma + ln_beta
    a = _smooth_leaky(a)
    al = jnp.einsum('bik,ik->bi', a, alpha_dot)
    mx = jax.ops.segment_max(al, dst, num_segments=N)
    ex = jnp.exp(al - mx[dst])
    den = jax.ops.segment_sum(ex, dst, num_segments=N)
    al = ex / (den[dst] + 1e-16)
    attn = msg.reshape(E, _NCOEFF, NH, VC) * al[:, None, :, None]
    attn = attn.reshape(E, _NCOEFF, NH * VC)
    attn = jnp.einsum('eij,ejc->eic', wigner_inv, attn)
    node = jax.ops.segment_sum(attn, dst, num_segments=N)
    return _so3_project(node, proj_w, proj_b)
